# baseline (speedup 1.0000x reference)
import jax
import jax.numpy as jnp
from jax import lax
from jax.experimental import pallas as pl
from jax.experimental.pallas import tpu as pltpu
from jax.experimental.pallas import tpu_sc as plsc

N = 10000
E = 320000
D = 128
DH = 256
DHALF = D // 2

NC = 2
NS = 16
L = 16

NPAD = 10240
ROWS_PER_TILE = NPAD // NS
CB = 128
NCHUNK = 160
EW = NCHUNK * CB
EPAD = NS * EW
BR = 256


def _sc_mesh():
    return plsc.VectorSubcoreMesh(
        core_axis_name="c", subcore_axis_name="s", num_cores=NC, num_subcores=NS
    )


def _deg_body(dst_hbm, out_hbm, dst_v, ones_v, zbuf, acc_sh):
    c = lax.axis_index("c")
    s = lax.axis_index("s")
    row0 = s * ROWS_PER_TILE

    ones = jnp.ones((L,), jnp.float32)
    zero = jnp.zeros((L,), jnp.float32)

    def fill(r, _):
        ones_v[r, :] = ones
        zbuf[r, :] = zero
        return ()

    lax.fori_loop(0, CB, fill, ())

    for k in range(ROWS_PER_TILE // CB):
        pltpu.sync_copy(zbuf, acc_sh.at[pl.ds(row0 + k * CB, CB)])
    pltpu.sync_copy(dst_hbm.at[s, pl.ds(c * (NCHUNK // 2), NCHUNK // 2)], dst_v)
    plsc.subcore_barrier()

    def body(j, _):
        pltpu.sync_copy(ones_v, acc_sh.at[dst_v.at[j]], add=True)
        return ()

    lax.fori_loop(0, NCHUNK // 2, body, ())
    plsc.subcore_barrier()
    pltpu.sync_copy(
        acc_sh.at[pl.ds(row0, ROWS_PER_TILE)],
        out_hbm.at[c, pl.ds(row0, ROWS_PER_TILE)],
    )


def _deg_counts(dst3d):
    k = pl.kernel(
        _deg_body,
        out_type=jax.ShapeDtypeStruct((NC, NPAD, L), jnp.float32),
        mesh=_sc_mesh(),
        scratch_types=[
            pltpu.VMEM((NCHUNK // 2, CB), jnp.int32),
            pltpu.VMEM((CB, L), jnp.float32),
            pltpu.VMEM((CB, L), jnp.float32),
            pltpu.VMEM_SHARED((NPAD, L), jnp.float32),
        ],
        compiler_params=pltpu.CompilerParams(use_tc_tiling_on_sc=False),
    )
    return k(dst3d)


RING = 4
NG = NCHUNK // RING


def _edge_body(y_hbm, src_hbm, dst_hbm, out_hbm, *scr):
    sw = scr[0:2]
    dw = scr[2:4]
    bufs = scr[4:4 + RING]
    sg = scr[4 + RING:4 + 2 * RING]
    ss = scr[4 + 2 * RING:4 + 3 * RING]
    siw = scr[4 + 3 * RING:6 + 3 * RING]
    table = scr[6 + 3 * RING]
    acc_sh = scr[7 + 3 * RING]

    c = lax.axis_index("c")
    s = lax.axis_index("s")
    row0 = s * ROWS_PER_TILE

    def load_win(g, p):
        pltpu.async_copy(src_hbm.at[s, pl.ds(g * RING, RING)], sw[p], siw[p])
        pltpu.async_copy(dst_hbm.at[s, pl.ds(g * RING, RING)], dw[p], siw[p])

    def wait_win(p):
        pltpu.make_async_copy(src_hbm.at[s, pl.ds(0, RING)], sw[p], siw[p]).wait()
        pltpu.make_async_copy(dst_hbm.at[s, pl.ds(0, RING)], dw[p], siw[p]).wait()

    load_win(0, 0)
    load_win(1, 1)
    pltpu.sync_copy(y_hbm.at[c, pl.ds(row0, ROWS_PER_TILE)],
                    table.at[pl.ds(row0, ROWS_PER_TILE)])

    zero = jnp.zeros((L,), jnp.float32)

    def zfill(r, _):
        for cc in range(DHALF // L):
            bufs[0][r, pl.ds(cc * L, L)] = zero
        return ()

    lax.fori_loop(0, CB, zfill, ())
    for k in range(ROWS_PER_TILE // CB):
        pltpu.sync_copy(bufs[0], acc_sh.at[pl.ds(row0 + k * CB, CB)])
    plsc.subcore_barrier()

    wait_win(0)
    for b in range(RING):
        pltpu.async_copy(table.at[sw[0].at[b]], bufs[b], sg[b])

    def body(t, _):
        for p in range(2):
            g = t * 2 + p
            gnext = g + 1
            pnext = 1 - p
            for b in range(RING):
                pltpu.make_async_copy(table.at[sw[p].at[b]], bufs[b], sg[b]).wait()
                pltpu.async_copy(bufs[b], acc_sh.at[dw[p].at[b]], ss[b], add=True)

            @pl.when(gnext < NG)
            def _():
                wait_win(pnext)

            for b in range(RING):
                pltpu.make_async_copy(bufs[b], acc_sh.at[dw[p].at[0]], ss[b]).wait()

                @pl.when(gnext < NG)
                def _():
                    pltpu.async_copy(table.at[sw[pnext].at[b]], bufs[b], sg[b])

            @pl.when(g + 2 < NG)
            def _():
                load_win(g + 2, p)

        return ()

    lax.fori_loop(0, NG // 2, body, ())
    plsc.subcore_barrier()
    for k in range(ROWS_PER_TILE // CB):
        pltpu.sync_copy(
            acc_sh.at[pl.ds(row0 + k * CB, CB)],
            out_hbm.at[c, pl.ds(row0 + k * CB, CB)],
        )


def _edge_pass(y_split, src3d, dst3d):
    k = pl.kernel(
        _edge_body,
        out_type=jax.ShapeDtypeStruct((NC, NPAD, DHALF), jnp.float32),
        mesh=_sc_mesh(),
        scratch_types=(
            [pltpu.VMEM((RING, CB), jnp.int32) for _ in range(4)]
            + [pltpu.VMEM((CB, DHALF), jnp.float32) for _ in range(RING)]
            + [pltpu.SemaphoreType.DMA for _ in range(2 * RING + 2)]
            + [pltpu.VMEM_SHARED((NPAD, DHALF), jnp.float32)]
            + [pltpu.VMEM_SHARED((NPAD, DHALF), jnp.float32)]
        ),
        compiler_params=pltpu.CompilerParams(use_tc_tiling_on_sc=False),
    )
    return k(y_split, src3d, dst3d)


def _prep_body(counts_ref, x_ref, y_ref, dis_ref):
    cb = counts_ref[...]
    deg = 1.0 + cb[0, :, 0:1] + cb[1, :, 0:1]
    dis = lax.rsqrt(deg)
    dis_ref[...] = dis
    y = x_ref[...] * dis
    y_ref[0] = y[:, :DHALF]
    y_ref[1] = y[:, DHALF:]


def _prep(counts, x_pad):
    return pl.pallas_call(
        _prep_body,
        grid=(NPAD // BR,),
        in_specs=[
            pl.BlockSpec((NC, BR, L), lambda i: (0, i, 0)),
            pl.BlockSpec((BR, D), lambda i: (i, 0)),
        ],
        out_specs=[
            pl.BlockSpec((2, BR, DHALF), lambda i: (0, i, 0)),
            pl.BlockSpec((BR, 1), lambda i: (i, 0)),
        ],
        out_shape=[
            jax.ShapeDtypeStruct((2, NPAD, DHALF), jnp.float32),
            jax.ShapeDtypeStruct((NPAD, 1), jnp.float32),
        ],
    )(counts, x_pad)


def _mid_body(s1_ref, y1_ref, dis_ref, w1_ref, b1_ref, w2_ref, y2_ref):
    dis = dis_ref[...]
    agg_l = (s1_ref[0] + y1_ref[0]) * dis
    agg_r = (s1_ref[1] + y1_ref[1]) * dis
    agg = jnp.concatenate([agg_l, agg_r], axis=1)
    h1 = jnp.dot(agg, w1_ref[...], preferred_element_type=jnp.float32,
                 precision=lax.Precision.HIGHEST)
    h1 = jnp.maximum(h1 + b1_ref[...], 0.0)
    p = jnp.dot(h1, w2_ref[...], preferred_element_type=jnp.float32,
                precision=lax.Precision.HIGHEST) * dis
    y2_ref[0] = p[:, :DHALF]
    y2_ref[1] = p[:, DHALF:]


def _mid(s1, y1, dis, W1, b1, W2):
    return pl.pallas_call(
        _mid_body,
        grid=(NPAD // BR,),
        in_specs=[
            pl.BlockSpec((2, BR, DHALF), lambda i: (0, i, 0)),
            pl.BlockSpec((2, BR, DHALF), lambda i: (0, i, 0)),
            pl.BlockSpec((BR, 1), lambda i: (i, 0)),
            pl.BlockSpec((D, DH), lambda i: (0, 0)),
            pl.BlockSpec((1, DH), lambda i: (0, 0)),
            pl.BlockSpec((DH, D), lambda i: (0, 0)),
        ],
        out_specs=pl.BlockSpec((2, BR, DHALF), lambda i: (0, i, 0)),
        out_shape=jax.ShapeDtypeStruct((2, NPAD, DHALF), jnp.float32),
    )(s1, y1, dis, W1, b1.reshape(1, DH), W2)


def _final_body(s2_ref, y2_ref, dis_ref, b2_ref, out_ref):
    dis = dis_ref[...]
    agg_l = (s2_ref[0] + y2_ref[0]) * dis
    agg_r = (s2_ref[1] + y2_ref[1]) * dis
    agg = jnp.concatenate([agg_l, agg_r], axis=1)
    out_ref[...] = jnp.maximum(agg + b2_ref[...], 0.0)


def _final(s2, y2, dis, b2):
    return pl.pallas_call(
        _final_body,
        grid=(NPAD // BR,),
        in_specs=[
            pl.BlockSpec((2, BR, DHALF), lambda i: (0, i, 0)),
            pl.BlockSpec((2, BR, DHALF), lambda i: (0, i, 0)),
            pl.BlockSpec((BR, 1), lambda i: (i, 0)),
            pl.BlockSpec((1, D), lambda i: (0, 0)),
        ],
        out_specs=pl.BlockSpec((BR, D), lambda i: (i, 0)),
        out_shape=jax.ShapeDtypeStruct((NPAD, D), jnp.float32),
    )(s2, y2, dis, b2.reshape(1, D))


def kernel(x, edge_index, W1, b1, W2, b2):
    src = edge_index[0]
    dst = edge_index[1]
    pad = EPAD - E
    padv = jnp.full((pad,), N, jnp.int32)
    src3d = jnp.concatenate([src, padv]).reshape(NS, NCHUNK, CB)
    dst3d = jnp.concatenate([dst, padv]).reshape(NS, NCHUNK, CB)
    x_pad = jnp.pad(x, ((0, NPAD - N), (0, 0)))

    counts = _deg_counts(dst3d)
    y1, dis = _prep(counts, x_pad)
    s1 = _edge_pass(y1, src3d, dst3d)
    y2 = _mid(s1, y1, dis, W1, b1, W2)
    s2 = _edge_pass(y2, src3d, dst3d)
    out = _final(s2, y2, dis, b2)
    return out[:N]


# speedup vs baseline: 1.4349x; 1.4349x over previous
import jax
import jax.numpy as jnp
from jax import lax
from jax.experimental import pallas as pl
from jax.experimental.pallas import tpu as pltpu
from jax.experimental.pallas import tpu_sc as plsc

N = 10000
E = 320000
D = 128
DH = 256
DHALF = D // 2

NC = 2
NS = 16
L = 16

NPAD = 10240
ROWS_PER_TILE = NPAD // NS
CB = 128
NCHUNK = 160
EW = NCHUNK * CB
EPAD = NS * EW
BR = 256


def _sc_mesh():
    return plsc.VectorSubcoreMesh(
        core_axis_name="c", subcore_axis_name="s", num_cores=NC, num_subcores=NS
    )


def _deg_body(dst_hbm, out_hbm, dst_v, ones_v, zbuf, acc_sh):
    c = lax.axis_index("c")
    s = lax.axis_index("s")
    row0 = s * ROWS_PER_TILE

    ones = jnp.ones((L,), jnp.float32)
    zero = jnp.zeros((L,), jnp.float32)

    def fill(r, _):
        ones_v[r, :] = ones
        zbuf[r, :] = zero
        return ()

    lax.fori_loop(0, CB, fill, ())

    for k in range(ROWS_PER_TILE // CB):
        pltpu.sync_copy(zbuf, acc_sh.at[pl.ds(row0 + k * CB, CB)])
    pltpu.sync_copy(dst_hbm.at[s, pl.ds(c * (NCHUNK // 2), NCHUNK // 2)], dst_v)
    plsc.subcore_barrier()

    def body(j, _):
        pltpu.sync_copy(ones_v, acc_sh.at[dst_v.at[j]], add=True)
        return ()

    lax.fori_loop(0, NCHUNK // 2, body, ())
    plsc.subcore_barrier()
    pltpu.sync_copy(
        acc_sh.at[pl.ds(row0, ROWS_PER_TILE)],
        out_hbm.at[c, pl.ds(row0, ROWS_PER_TILE)],
    )


def _deg_counts(dst3d):
    k = pl.kernel(
        _deg_body,
        out_type=jax.ShapeDtypeStruct((NC, NPAD, L), jnp.float32),
        mesh=_sc_mesh(),
        scratch_types=[
            pltpu.VMEM((NCHUNK // 2, CB), jnp.int32),
            pltpu.VMEM((CB, L), jnp.float32),
            pltpu.VMEM((CB, L), jnp.float32),
            pltpu.VMEM_SHARED((NPAD, L), jnp.float32),
        ],
        compiler_params=pltpu.CompilerParams(use_tc_tiling_on_sc=False),
    )
    return k(dst3d)


RING = 4
NG = NCHUNK // RING


def _edge_body(y_hbm, src_hbm, dst_hbm, out_hbm, *scr):
    sw = scr[0:2]
    dw = scr[2:4]
    bufs = scr[4:4 + RING]
    sg = scr[4 + RING:4 + 2 * RING]
    ss = scr[4 + 2 * RING:4 + 3 * RING]
    siw = scr[4 + 3 * RING:6 + 3 * RING]
    table = scr[6 + 3 * RING]
    acc_sh = scr[7 + 3 * RING]

    c = lax.axis_index("c")
    s = lax.axis_index("s")
    row0 = s * ROWS_PER_TILE

    def load_win(g, p):
        pltpu.async_copy(src_hbm.at[s, pl.ds(g * RING, RING)], sw[p], siw[p])
        pltpu.async_copy(dst_hbm.at[s, pl.ds(g * RING, RING)], dw[p], siw[p])

    def wait_win(p):
        pltpu.make_async_copy(src_hbm.at[s, pl.ds(0, RING)], sw[p], siw[p]).wait()
        pltpu.make_async_copy(dst_hbm.at[s, pl.ds(0, RING)], dw[p], siw[p]).wait()

    load_win(0, 0)
    load_win(1, 1)
    pltpu.sync_copy(y_hbm.at[c, pl.ds(row0, ROWS_PER_TILE)],
                    table.at[pl.ds(row0, ROWS_PER_TILE)])

    zero = jnp.zeros((2 * L,), jnp.bfloat16)

    def zfill(r, _):
        for cc in range(DHALF // (2 * L)):
            bufs[0][r, pl.ds(cc * 2 * L, 2 * L)] = zero
        return ()

    lax.fori_loop(0, CB, zfill, ())
    for k in range(ROWS_PER_TILE // CB):
        pltpu.sync_copy(bufs[0], acc_sh.at[pl.ds(row0 + k * CB, CB)])
    plsc.subcore_barrier()

    wait_win(0)
    for b in range(RING):
        pltpu.async_copy(table.at[sw[0].at[b]], bufs[b], sg[b])

    def body(t, _):
        for p in range(2):
            g = t * 2 + p
            gnext = g + 1
            pnext = 1 - p
            for b in range(RING):
                pltpu.make_async_copy(table.at[sw[p].at[b]], bufs[b], sg[b]).wait()
                pltpu.async_copy(bufs[b], acc_sh.at[dw[p].at[b]], ss[b], add=True)

            @pl.when(gnext < NG)
            def _():
                wait_win(pnext)

            for b in range(RING):
                pltpu.make_async_copy(bufs[b], acc_sh.at[dw[p].at[0]], ss[b]).wait()

                @pl.when(gnext < NG)
                def _():
                    pltpu.async_copy(table.at[sw[pnext].at[b]], bufs[b], sg[b])

            @pl.when(g + 2 < NG)
            def _():
                load_win(g + 2, p)

        return ()

    lax.fori_loop(0, NG // 2, body, ())
    plsc.subcore_barrier()
    for k in range(ROWS_PER_TILE // CB):
        pltpu.sync_copy(
            acc_sh.at[pl.ds(row0 + k * CB, CB)],
            out_hbm.at[c, pl.ds(row0 + k * CB, CB)],
        )


def _edge_pass(y_split, src3d, dst3d):
    k = pl.kernel(
        _edge_body,
        out_type=jax.ShapeDtypeStruct((NC, NPAD, DHALF), jnp.bfloat16),
        mesh=_sc_mesh(),
        scratch_types=(
            [pltpu.VMEM((RING, CB), jnp.int32) for _ in range(4)]
            + [pltpu.VMEM((CB, DHALF), jnp.bfloat16) for _ in range(RING)]
            + [pltpu.SemaphoreType.DMA for _ in range(2 * RING + 2)]
            + [pltpu.VMEM_SHARED((NPAD, DHALF), jnp.bfloat16)]
            + [pltpu.VMEM_SHARED((NPAD, DHALF), jnp.bfloat16)]
        ),
        compiler_params=pltpu.CompilerParams(use_tc_tiling_on_sc=False),
    )
    return k(y_split, src3d, dst3d)


def _prep_body(counts_ref, x_ref, y_ref, dis_ref):
    cb = counts_ref[...]
    deg = 1.0 + cb[0, :, 0:1] + cb[1, :, 0:1]
    dis = lax.rsqrt(deg)
    dis_ref[...] = dis
    y = (x_ref[...] * dis).astype(jnp.bfloat16)
    y_ref[0] = y[:, :DHALF]
    y_ref[1] = y[:, DHALF:]


def _prep(counts, x_pad):
    return pl.pallas_call(
        _prep_body,
        grid=(NPAD // BR,),
        in_specs=[
            pl.BlockSpec((NC, BR, L), lambda i: (0, i, 0)),
            pl.BlockSpec((BR, D), lambda i: (i, 0)),
        ],
        out_specs=[
            pl.BlockSpec((2, BR, DHALF), lambda i: (0, i, 0)),
            pl.BlockSpec((BR, 1), lambda i: (i, 0)),
        ],
        out_shape=[
            jax.ShapeDtypeStruct((2, NPAD, DHALF), jnp.bfloat16),
            jax.ShapeDtypeStruct((NPAD, 1), jnp.float32),
        ],
    )(counts, x_pad)


def _mid_body(s1_ref, y1_ref, dis_ref, w1_ref, b1_ref, w2_ref, y2_ref):
    dis = dis_ref[...]
    agg_l = (s1_ref[0].astype(jnp.float32) + y1_ref[0].astype(jnp.float32)) * dis
    agg_r = (s1_ref[1].astype(jnp.float32) + y1_ref[1].astype(jnp.float32)) * dis
    agg = jnp.concatenate([agg_l, agg_r], axis=1)
    h1 = jnp.dot(agg, w1_ref[...], preferred_element_type=jnp.float32,
                 precision=lax.Precision.HIGHEST)
    h1 = jnp.maximum(h1 + b1_ref[...], 0.0)
    p = jnp.dot(h1, w2_ref[...], preferred_element_type=jnp.float32,
                precision=lax.Precision.HIGHEST) * dis
    pb = p.astype(jnp.bfloat16)
    y2_ref[0] = pb[:, :DHALF]
    y2_ref[1] = pb[:, DHALF:]


def _mid(s1, y1, dis, W1, b1, W2):
    return pl.pallas_call(
        _mid_body,
        grid=(NPAD // BR,),
        in_specs=[
            pl.BlockSpec((2, BR, DHALF), lambda i: (0, i, 0)),
            pl.BlockSpec((2, BR, DHALF), lambda i: (0, i, 0)),
            pl.BlockSpec((BR, 1), lambda i: (i, 0)),
            pl.BlockSpec((D, DH), lambda i: (0, 0)),
            pl.BlockSpec((1, DH), lambda i: (0, 0)),
            pl.BlockSpec((DH, D), lambda i: (0, 0)),
        ],
        out_specs=pl.BlockSpec((2, BR, DHALF), lambda i: (0, i, 0)),
        out_shape=jax.ShapeDtypeStruct((2, NPAD, DHALF), jnp.bfloat16),
    )(s1, y1, dis, W1, b1.reshape(1, DH), W2)


def _final_body(s2_ref, y2_ref, dis_ref, b2_ref, out_ref):
    dis = dis_ref[...]
    agg_l = (s2_ref[0].astype(jnp.float32) + y2_ref[0].astype(jnp.float32)) * dis
    agg_r = (s2_ref[1].astype(jnp.float32) + y2_ref[1].astype(jnp.float32)) * dis
    agg = jnp.concatenate([agg_l, agg_r], axis=1)
    out_ref[...] = jnp.maximum(agg + b2_ref[...], 0.0)


def _final(s2, y2, dis, b2):
    return pl.pallas_call(
        _final_body,
        grid=(NPAD // BR,),
        in_specs=[
            pl.BlockSpec((2, BR, DHALF), lambda i: (0, i, 0)),
            pl.BlockSpec((2, BR, DHALF), lambda i: (0, i, 0)),
            pl.BlockSpec((BR, 1), lambda i: (i, 0)),
            pl.BlockSpec((1, D), lambda i: (0, 0)),
        ],
        out_specs=pl.BlockSpec((BR, D), lambda i: (i, 0)),
        out_shape=jax.ShapeDtypeStruct((NPAD, D), jnp.float32),
    )(s2, y2, dis, b2.reshape(1, D))


def kernel(x, edge_index, W1, b1, W2, b2):
    src = edge_index[0]
    dst = edge_index[1]
    pad = EPAD - E
    padv = jnp.full((pad,), N, jnp.int32)
    src3d = jnp.concatenate([src, padv]).reshape(NS, NCHUNK, CB)
    dst3d = jnp.concatenate([dst, padv]).reshape(NS, NCHUNK, CB)
    x_pad = jnp.pad(x, ((0, NPAD - N), (0, 0)))

    counts = _deg_counts(dst3d)
    y1, dis = _prep(counts, x_pad)
    s1 = _edge_pass(y1, src3d, dst3d)
    y2 = _mid(s1, y1, dis, W1, b1, W2)
    s2 = _edge_pass(y2, src3d, dst3d)
    out = _final(s2, y2, dis, b2)
    return out[:N]
